# baseline (speedup 1.0000x reference)
import functools

import jax
import jax.numpy as jnp
import numpy as np
from jax.experimental import pallas as pl
from jax.experimental.pallas import tpu as pltpu

_TRANS_WEIGHT = 1.0
_ROT_WEIGHT = 100.0
_PI = np.float32(np.pi)
_TWO_PI = np.float32(2.0 * np.pi)


def _loss_kernel(const_ref, p_ref, t_ref, out_ref, acc_ref, *, n_steps, inv_n):
    j = pl.program_id(0)

    @pl.when(j == 0)
    def _():
        acc_ref[...] = jnp.zeros_like(acc_ref)

    hi = const_ref[0:1, :]
    lo = const_ref[1:2, :]
    p = p_ref[...]
    t = t_ref[...]
    pb = jnp.where(p > hi, p - _TWO_PI, p)
    pn = jnp.where(pb < lo, pb + _TWO_PI, pb)
    tb = jnp.where(t > hi, t - _TWO_PI, t)
    tn = jnp.where(tb < lo, tb + _TWO_PI, tb)
    d = pn - tn
    acc_ref[...] += jnp.sum(d * d, axis=0, keepdims=True)

    @pl.when(j == n_steps - 1)
    def _():
        acc = acc_ref[...]
        trans_mask = const_ref[2:3, :]
        s_trans = jnp.sum(acc * trans_mask)
        s_all = jnp.sum(acc)
        trans_loss = s_trans * inv_n * _TRANS_WEIGHT
        rot_loss = (s_all - s_trans) * inv_n * _ROT_WEIGHT
        out_ref[0, 0] = trans_loss + rot_loss
        out_ref[0, 1] = trans_loss
        out_ref[0, 2] = rot_loss


def kernel(pred, target, *, interpret=False):
    B, T, D = pred.shape
    C = T * D
    BB = 2048
    G = B // BB
    p2 = pred.reshape(B, C)
    t2 = target.reshape(B, C)

    d_idx = np.arange(C) % D
    is_rot = d_idx >= 3
    hi = np.where(is_rot, _PI, np.inf).astype(np.float32)
    lo = np.where(is_rot, -_PI, -np.inf).astype(np.float32)
    trans_mask = (~is_rot).astype(np.float32)
    const = jnp.asarray(np.stack([hi, lo, trans_mask]))

    n_per_half = B * T * 3
    out = pl.pallas_call(
        functools.partial(
            _loss_kernel, n_steps=G, inv_n=np.float32(1.0 / n_per_half)
        ),
        grid=(G,),
        in_specs=[
            pl.BlockSpec((3, C), lambda j: (0, 0)),
            pl.BlockSpec((BB, C), lambda j: (j, 0)),
            pl.BlockSpec((BB, C), lambda j: (j, 0)),
        ],
        out_specs=pl.BlockSpec(memory_space=pltpu.SMEM),
        out_shape=jax.ShapeDtypeStruct((1, 3), jnp.float32),
        scratch_shapes=[pltpu.VMEM((1, C), jnp.float32)],
        compiler_params=pltpu.CompilerParams(
            dimension_semantics=("arbitrary",),
        ),
        name="weighted_mse_loss",
        interpret=interpret,
    )(const, p2, t2)

    return (out[0, 0], out[0, 1], out[0, 2])


# speedup vs baseline: 1.0655x; 1.0655x over previous
import functools

import jax
import jax.numpy as jnp
import numpy as np
from jax.experimental import pallas as pl
from jax.experimental.pallas import tpu as pltpu

_TRANS_WEIGHT = 1.0
_ROT_WEIGHT = 100.0
_PI = np.float32(np.pi)
_TWO_PI = np.float32(2.0 * np.pi)


def _wrap_correction(a, hi, lo):
    c = jnp.where(a > hi, jnp.float32(-_TWO_PI), jnp.float32(0.0))
    return jnp.where(a < lo, jnp.float32(_TWO_PI), c)


def _loss_kernel(const_ref, p_ref, t_ref, out_ref, acc_ref, *, n_steps, bb,
                 inv_n):
    j = pl.program_id(0)
    c_shape = (8, acc_ref.shape[1])

    hi = const_ref[0:8, :]
    lo = const_ref[8:16, :]

    acc = jnp.zeros(c_shape, jnp.float32)
    for i in range(bb // 8):
        p = p_ref[i * 8:(i + 1) * 8, :]
        t = t_ref[i * 8:(i + 1) * 8, :]
        d = (p - t) + (_wrap_correction(p, hi, lo) - _wrap_correction(t, hi, lo))
        acc = acc + d * d

    @pl.when(j == 0)
    def _():
        acc_ref[...] = acc

    @pl.when(j > 0)
    def _():
        acc_ref[...] += acc

    @pl.when(j == n_steps - 1)
    def _():
        total = acc_ref[...]
        trans_mask = const_ref[16:24, :]
        s_trans = jnp.sum(total * trans_mask)
        s_all = jnp.sum(total)
        trans_loss = s_trans * inv_n * _TRANS_WEIGHT
        rot_loss = (s_all - s_trans) * inv_n * _ROT_WEIGHT
        out_ref[0, 0] = trans_loss + rot_loss
        out_ref[0, 1] = trans_loss
        out_ref[0, 2] = rot_loss


def kernel(pred, target, *, interpret=False):
    B, T, D = pred.shape
    C = T * D
    BB = 1024
    G = B // BB
    p2 = pred.reshape(B, C)
    t2 = target.reshape(B, C)

    d_idx = np.arange(C) % D
    is_rot = d_idx >= 3
    hi = np.where(is_rot, _PI, np.inf).astype(np.float32)
    lo = np.where(is_rot, -_PI, -np.inf).astype(np.float32)
    trans_mask = (~is_rot).astype(np.float32)
    const = jnp.asarray(
        np.concatenate([
            np.tile(hi, (8, 1)),
            np.tile(lo, (8, 1)),
            np.tile(trans_mask, (8, 1)),
        ])
    )

    n_per_half = B * T * 3
    out = pl.pallas_call(
        functools.partial(
            _loss_kernel, n_steps=G, bb=BB, inv_n=np.float32(1.0 / n_per_half)
        ),
        grid=(G,),
        in_specs=[
            pl.BlockSpec((24, C), lambda j: (0, 0)),
            pl.BlockSpec((BB, C), lambda j: (j, 0)),
            pl.BlockSpec((BB, C), lambda j: (j, 0)),
        ],
        out_specs=pl.BlockSpec(memory_space=pltpu.SMEM),
        out_shape=jax.ShapeDtypeStruct((1, 3), jnp.float32),
        scratch_shapes=[pltpu.VMEM((8, C), jnp.float32)],
        compiler_params=pltpu.CompilerParams(
            dimension_semantics=("arbitrary",),
        ),
        name="weighted_mse_loss",
        interpret=interpret,
    )(const, p2, t2)

    return (out[0, 0], out[0, 1], out[0, 2])
